# Initial kernel scaffold; baseline (speedup 1.0000x reference)
import jax
import jax.numpy as jnp
from jax.experimental import pallas as pl


def kernel(last_hidden_states, W_dist, b_dist, centroids, U, V, Ug, Vg, bg):
    raise NotImplementedError("write your pallas kernel here")


# speedup vs baseline: 2.6674x; 2.6674x over previous
import functools

import jax
import jax.numpy as jnp
from jax.experimental import pallas as pl
from jax.experimental.pallas import tpu as pltpu

NUM_EXPERTS = 8
TOP_K = 2
HIDDEN = 1024
TOPIC = 128
RANK = 32
TOKENS = 8192

BLOCK = 512


def _moe_block_kernel(x_ref, wd_ref, bd_ref, c_ref, u_ref, v_ref, ug_ref,
                      vg_ref, bg_ref, out_ref):
    x = x_ref[...]

    distilled = jax.nn.gelu(
        jnp.dot(x, wd_ref[...], preferred_element_type=jnp.float32)
        + bd_ref[...])
    dn = distilled / jnp.maximum(
        jnp.sqrt(jnp.sum(distilled * distilled, axis=-1, keepdims=True)), 1e-8)
    c = c_ref[...]
    cn = c / jnp.maximum(
        jnp.sqrt(jnp.sum(c * c, axis=-1, keepdims=True)), 1e-8)
    d2 = (jnp.sum(dn * dn, axis=-1, keepdims=True)
          + jnp.sum(cn * cn, axis=-1)[None, :]
          - 2.0 * jnp.dot(dn, cn.T, preferred_element_type=jnp.float32))
    dist = jnp.sqrt(jnp.maximum(d2, 0.0))
    neg = -dist
    m = jnp.max(neg, axis=-1, keepdims=True)
    e = jnp.exp(neg - m)
    p = e / jnp.sum(e, axis=-1, keepdims=True)

    eidx = jax.lax.broadcasted_iota(jnp.int32, p.shape, 1)
    i1 = jnp.argmax(p, axis=-1)[:, None]
    p1 = jnp.max(p, axis=-1, keepdims=True)
    masked = jnp.where(eidx == i1, -jnp.inf, p)
    i2 = jnp.argmax(masked, axis=-1)[:, None]
    p2 = jnp.max(masked, axis=-1, keepdims=True)
    w = jnp.where(eidx == i1, p1, 0.0) + jnp.where(eidx == i2, p2, 0.0)

    acc = jnp.zeros_like(x)
    for ei in range(NUM_EXPERTS):
        h = jnp.dot(jnp.dot(x, u_ref[ei], preferred_element_type=jnp.float32),
                    v_ref[ei], preferred_element_type=jnp.float32)
        g = jax.nn.sigmoid(
            jnp.dot(jnp.dot(x, ug_ref[ei], preferred_element_type=jnp.float32),
                    vg_ref[ei], preferred_element_type=jnp.float32)
            + bg_ref[ei][None, :])
        we = w[:, ei][:, None]
        acc = acc + we * (g * jnp.maximum(h, 0.0) + (1.0 - g) * x)
    out_ref[...] = acc


@jax.jit
def kernel(last_hidden_states, W_dist, b_dist, centroids, U, V, Ug, Vg, bg):
    n = last_hidden_states.shape[0]
    grid = (n // BLOCK,)
    full = lambda shape: pl.BlockSpec(shape, lambda i: (0,) * len(shape))
    return pl.pallas_call(
        _moe_block_kernel,
        grid=grid,
        in_specs=[
            pl.BlockSpec((BLOCK, HIDDEN), lambda i: (i, 0)),
            full((HIDDEN, TOPIC)),
            full((TOPIC,)),
            full((NUM_EXPERTS, TOPIC)),
            full((NUM_EXPERTS, HIDDEN, RANK)),
            full((NUM_EXPERTS, RANK, HIDDEN)),
            full((NUM_EXPERTS, HIDDEN, RANK)),
            full((NUM_EXPERTS, RANK, HIDDEN)),
            full((NUM_EXPERTS, HIDDEN)),
        ],
        out_specs=pl.BlockSpec((BLOCK, HIDDEN), lambda i: (i, 0)),
        out_shape=jax.ShapeDtypeStruct((n, HIDDEN), jnp.float32),
    )(last_hidden_states, W_dist, b_dist, centroids, U, V, Ug, Vg, bg)


# speedup vs baseline: 3.4176x; 1.2812x over previous
import functools

import jax
import jax.numpy as jnp
from jax.experimental import pallas as pl
from jax.experimental.pallas import tpu as pltpu

NUM_EXPERTS = 8
TOP_K = 2
HIDDEN = 1024
TOPIC = 128
RANK = 32
TOKENS = 8192

BLOCK = 512


def _moe_block_kernel(x_ref, wd_ref, bd_ref, c_ref, uu_ref, v_ref, vg_ref,
                      bg_ref, out_ref):
    x = x_ref[...]

    distilled = jax.nn.gelu(
        jnp.dot(x, wd_ref[...], preferred_element_type=jnp.float32)
        + bd_ref[...])
    dn = distilled / jnp.maximum(
        jnp.sqrt(jnp.sum(distilled * distilled, axis=-1, keepdims=True)), 1e-8)
    c = c_ref[...]
    cn = c / jnp.maximum(
        jnp.sqrt(jnp.sum(c * c, axis=-1, keepdims=True)), 1e-8)
    d2 = (jnp.sum(dn * dn, axis=-1, keepdims=True)
          + jnp.sum(cn * cn, axis=-1)[None, :]
          - 2.0 * jnp.dot(dn, cn.T, preferred_element_type=jnp.float32))
    dist = jnp.sqrt(jnp.maximum(d2, 0.0))
    neg = -dist
    m = jnp.max(neg, axis=-1, keepdims=True)
    e = jnp.exp(neg - m)
    p = e / jnp.sum(e, axis=-1, keepdims=True)

    eidx = jax.lax.broadcasted_iota(jnp.int32, p.shape, 1)
    i1 = jnp.argmax(p, axis=-1)[:, None]
    p1 = jnp.max(p, axis=-1, keepdims=True)
    masked = jnp.where(eidx == i1, -jnp.inf, p)
    i2 = jnp.argmax(masked, axis=-1)[:, None]
    p2 = jnp.max(masked, axis=-1, keepdims=True)
    w = jnp.where(eidx == i1, p1, 0.0) + jnp.where(eidx == i2, p2, 0.0)

    xb = x.astype(jnp.bfloat16)
    r = jnp.dot(xb, uu_ref[...], preferred_element_type=jnp.float32)
    rb = r.astype(jnp.bfloat16)

    acc = jnp.zeros_like(x)
    for ei in range(NUM_EXPERTS):
        rh = rb[:, ei * RANK:(ei + 1) * RANK]
        rg = rb[:, (NUM_EXPERTS + ei) * RANK:(NUM_EXPERTS + ei + 1) * RANK]
        h = jnp.dot(rh, v_ref[ei], preferred_element_type=jnp.float32)
        g = jax.nn.sigmoid(
            jnp.dot(rg, vg_ref[ei], preferred_element_type=jnp.float32)
            + bg_ref[ei][None, :])
        we = w[:, ei][:, None]
        acc = acc + we * (g * jnp.maximum(h, 0.0) + (1.0 - g) * x)
    out_ref[...] = acc


@jax.jit
def kernel(last_hidden_states, W_dist, b_dist, centroids, U, V, Ug, Vg, bg):
    n = last_hidden_states.shape[0]
    uu = jnp.concatenate(
        [U.transpose(1, 0, 2).reshape(HIDDEN, NUM_EXPERTS * RANK),
         Ug.transpose(1, 0, 2).reshape(HIDDEN, NUM_EXPERTS * RANK)],
        axis=1).astype(jnp.bfloat16)
    vb = V.astype(jnp.bfloat16)
    vgb = Vg.astype(jnp.bfloat16)

    grid = (n // BLOCK,)
    full = lambda shape: pl.BlockSpec(shape, lambda i: (0,) * len(shape))
    return pl.pallas_call(
        _moe_block_kernel,
        grid=grid,
        in_specs=[
            pl.BlockSpec((BLOCK, HIDDEN), lambda i: (i, 0)),
            full((HIDDEN, TOPIC)),
            full((TOPIC,)),
            full((NUM_EXPERTS, TOPIC)),
            full((HIDDEN, 2 * NUM_EXPERTS * RANK)),
            full((NUM_EXPERTS, RANK, HIDDEN)),
            full((NUM_EXPERTS, RANK, HIDDEN)),
            full((NUM_EXPERTS, HIDDEN)),
        ],
        out_specs=pl.BlockSpec((BLOCK, HIDDEN), lambda i: (i, 0)),
        out_shape=jax.ShapeDtypeStruct((n, HIDDEN), jnp.float32),
    )(last_hidden_states, W_dist, b_dist, centroids, uu, vb, vgb, bg)


# speedup vs baseline: 3.7775x; 1.1053x over previous
import functools

import jax
import jax.numpy as jnp
from jax.experimental import pallas as pl
from jax.experimental.pallas import tpu as pltpu

NUM_EXPERTS = 8
TOP_K = 2
HIDDEN = 1024
TOPIC = 128
RANK = 32
TOKENS = 8192

BLOCK = 512


def _moe_block_kernel(x_ref, wd_ref, bd_ref, c_ref, uu_ref, v_ref, vg_ref,
                      bg_ref, out_ref):
    x = x_ref[...]

    distilled = jax.nn.gelu(
        jnp.dot(x, wd_ref[...], preferred_element_type=jnp.float32)
        + bd_ref[...])
    dn = distilled / jnp.maximum(
        jnp.sqrt(jnp.sum(distilled * distilled, axis=-1, keepdims=True)), 1e-8)
    c = c_ref[...]
    cn = c / jnp.maximum(
        jnp.sqrt(jnp.sum(c * c, axis=-1, keepdims=True)), 1e-8)
    d2 = (jnp.sum(dn * dn, axis=-1, keepdims=True)
          + jnp.sum(cn * cn, axis=-1)[None, :]
          - 2.0 * jnp.dot(dn, cn.T, preferred_element_type=jnp.float32))
    dist = jnp.sqrt(jnp.maximum(d2, 0.0))
    neg = -dist
    m = jnp.max(neg, axis=-1, keepdims=True)
    e = jnp.exp(neg - m)
    p = e / jnp.sum(e, axis=-1, keepdims=True)

    eidx = jax.lax.broadcasted_iota(jnp.int32, p.shape, 1)
    i1 = jnp.argmax(p, axis=-1)[:, None]
    p1 = jnp.max(p, axis=-1, keepdims=True)
    masked = jnp.where(eidx == i1, -jnp.inf, p)
    i2 = jnp.argmax(masked, axis=-1)[:, None]
    p2 = jnp.max(masked, axis=-1, keepdims=True)
    w = jnp.where(eidx == i1, p1, 0.0) + jnp.where(eidx == i2, p2, 0.0)

    xb = x.astype(jnp.bfloat16)
    r = jnp.dot(xb, uu_ref[...], preferred_element_type=jnp.float32)
    rb = r.astype(jnp.bfloat16)

    a = jnp.zeros_like(x)
    gsum = jnp.zeros_like(x)
    for ei in range(NUM_EXPERTS):
        we = w[:, ei][:, None]
        rh = (r[:, ei * RANK:(ei + 1) * RANK] * we).astype(jnp.bfloat16)
        rg = rb[:, (NUM_EXPERTS + ei) * RANK:(NUM_EXPERTS + ei + 1) * RANK]
        h = jnp.dot(rh, v_ref[ei], preferred_element_type=jnp.float32)
        g = jax.nn.sigmoid(
            jnp.dot(rg, vg_ref[ei], preferred_element_type=jnp.float32)
            + bg_ref[ei][None, :])
        a = a + g * jnp.maximum(h, 0.0)
        gsum = gsum + we * g
    wsum = p1 + p2
    out_ref[...] = a + (wsum - gsum) * x


@jax.jit
def kernel(last_hidden_states, W_dist, b_dist, centroids, U, V, Ug, Vg, bg):
    n = last_hidden_states.shape[0]
    uu = jnp.concatenate(
        [U.transpose(1, 0, 2).reshape(HIDDEN, NUM_EXPERTS * RANK),
         Ug.transpose(1, 0, 2).reshape(HIDDEN, NUM_EXPERTS * RANK)],
        axis=1).astype(jnp.bfloat16)
    vb = V.astype(jnp.bfloat16)
    vgb = Vg.astype(jnp.bfloat16)

    grid = (n // BLOCK,)
    full = lambda shape: pl.BlockSpec(shape, lambda i: (0,) * len(shape))
    return pl.pallas_call(
        _moe_block_kernel,
        grid=grid,
        in_specs=[
            pl.BlockSpec((BLOCK, HIDDEN), lambda i: (i, 0)),
            full((HIDDEN, TOPIC)),
            full((TOPIC,)),
            full((NUM_EXPERTS, TOPIC)),
            full((HIDDEN, 2 * NUM_EXPERTS * RANK)),
            full((NUM_EXPERTS, RANK, HIDDEN)),
            full((NUM_EXPERTS, RANK, HIDDEN)),
            full((NUM_EXPERTS, HIDDEN)),
        ],
        out_specs=pl.BlockSpec((BLOCK, HIDDEN), lambda i: (i, 0)),
        out_shape=jax.ShapeDtypeStruct((n, HIDDEN), jnp.float32),
    )(last_hidden_states, W_dist, b_dist, centroids, uu, vb, vgb, bg)
